# baseline (speedup 1.0000x reference)
import functools

import jax
import jax.numpy as jnp
from jax.experimental import pallas as pl
from jax.experimental.pallas import tpu as pltpu

_NSPLIT = 2


def _pick_block(n: int) -> int:
    for cand in (400, 200, 80, 40, 16, 8):
        if n % cand == 0:
            return cand
    return n


def _gcn_kernel(*refs, bi: int, nsplit: int):
    (x_ref, w1_ref, b1_ref, w2_ref, b2_ref), adj_refs = refs[:5], refs[5:5 + nsplit]
    o_ref, s1_scr, s2_scr = refs[5 + nsplit], refs[6 + nsplit], refs[7 + nsplit]
    p = pl.program_id(0)
    i = pl.program_id(1)
    bs = bi // nsplit

    @pl.when((p == 0) & (i == 0))
    def _():
        s1_scr[...] = jnp.dot(
            x_ref[...], w1_ref[...], preferred_element_type=jnp.float32
        )

    @pl.when(p == 0)
    def _():
        for j in range(nsplit):
            acc = jnp.dot(
                adj_refs[j][...], s1_scr[...], preferred_element_type=jnp.float32
            )
            h = jnp.maximum(acc + b1_ref[...], 0.0)
            s2 = jnp.dot(h, w2_ref[...], preferred_element_type=jnp.float32)
            s2_scr[pl.ds(i * bi + j * bs, bs), :] = s2
            o_ref[j * bs:(j + 1) * bs, :] = s2

    @pl.when(p == 1)
    def _():
        for j in range(nsplit):
            acc = jnp.dot(
                adj_refs[j][...], s2_scr[...], preferred_element_type=jnp.float32
            )
            o_ref[j * bs:(j + 1) * bs, :] = acc + b2_ref[...]


def kernel(x, adj, W1, b1, W2, b2):
    n, nfeat = x.shape
    nhid = W1.shape[1]
    nout = W2.shape[1]
    bi = _pick_block(n)
    nsplit = _NSPLIT
    bs = bi // nsplit

    b1r = b1.reshape(1, nhid)
    b2r = b2.reshape(1, nout)

    adj_specs = [
        pl.BlockSpec((bs, n), lambda p, i, j=j: (i * nsplit + j, 0))
        for j in range(nsplit)
    ]
    return pl.pallas_call(
        functools.partial(_gcn_kernel, bi=bi, nsplit=nsplit),
        grid=(2, n // bi),
        in_specs=[
            pl.BlockSpec((n, nfeat), lambda p, i: (0, 0)),
            pl.BlockSpec((nfeat, nhid), lambda p, i: (0, 0)),
            pl.BlockSpec((1, nhid), lambda p, i: (0, 0)),
            pl.BlockSpec((nhid, nout), lambda p, i: (0, 0)),
            pl.BlockSpec((1, nout), lambda p, i: (0, 0)),
        ] + adj_specs,
        out_specs=pl.BlockSpec((bi, nout), lambda p, i: (i, 0)),
        out_shape=jax.ShapeDtypeStruct((n, nout), jnp.float32),
        scratch_shapes=[
            pltpu.VMEM((n, nhid), jnp.float32),
            pltpu.VMEM((n, nhid), jnp.float32),
        ],
        compiler_params=pltpu.CompilerParams(
            dimension_semantics=("arbitrary", "arbitrary"),
        ),
    )(x, W1, b1r, W2, b2r, *([adj] * nsplit))


# speedup vs baseline: 1.0559x; 1.0559x over previous
import functools

import jax
import jax.numpy as jnp
from jax import lax
from jax.experimental import pallas as pl
from jax.experimental.pallas import tpu as pltpu

_BS = 80
_K = 6


def _gcn_kernel(
    x_ref, w1_ref, b1_ref, w2_ref, b2_ref, adj_hbm,
    o_ref, s1_scr, s2_scr, ring, sems, *, bs: int, nstep: int, k: int
):
    p = pl.program_id(0)
    i = pl.program_id(1)
    g = p * nstep + i

    def _issue(t):
        r = lax.rem(t, nstep)
        slot = lax.rem(t, k)
        pltpu.make_async_copy(
            adj_hbm.at[pl.ds(r * bs, bs), :], ring.at[slot], sems.at[slot]
        ).start()

    @pl.when(g == 0)
    def _():
        s1_scr[...] = jnp.dot(
            x_ref[...], w1_ref[...], preferred_element_type=jnp.float32
        )
        for t in range(k):
            _issue(t)

    @pl.when(g > 0)
    def _():
        t = g + k - 1

        @pl.when(t < 2 * nstep)
        def _():
            _issue(t)

    slot = lax.rem(g, k)
    r = lax.rem(g, nstep)
    pltpu.make_async_copy(
        adj_hbm.at[pl.ds(r * bs, bs), :], ring.at[slot], sems.at[slot]
    ).wait()
    a = ring[slot]

    @pl.when(p == 0)
    def _():
        acc = jnp.dot(a, s1_scr[...], preferred_element_type=jnp.float32)
        h = jnp.maximum(acc + b1_ref[...], 0.0)
        s2 = jnp.dot(h, w2_ref[...], preferred_element_type=jnp.float32)
        s2_scr[pl.ds(i * bs, bs), :] = s2
        o_ref[...] = s2

    @pl.when(p == 1)
    def _():
        acc = jnp.dot(a, s2_scr[...], preferred_element_type=jnp.float32)
        o_ref[...] = acc + b2_ref[...]


def kernel(x, adj, W1, b1, W2, b2):
    n, nfeat = x.shape
    nhid = W1.shape[1]
    nout = W2.shape[1]
    bs = _BS if n % _BS == 0 else n
    k = _K if n != bs else 1
    nstep = n // bs

    b1r = b1.reshape(1, nhid)
    b2r = b2.reshape(1, nout)

    return pl.pallas_call(
        functools.partial(_gcn_kernel, bs=bs, nstep=nstep, k=k),
        grid=(2, nstep),
        in_specs=[
            pl.BlockSpec((n, nfeat), lambda p, i: (0, 0)),
            pl.BlockSpec((nfeat, nhid), lambda p, i: (0, 0)),
            pl.BlockSpec((1, nhid), lambda p, i: (0, 0)),
            pl.BlockSpec((nhid, nout), lambda p, i: (0, 0)),
            pl.BlockSpec((1, nout), lambda p, i: (0, 0)),
            pl.BlockSpec(memory_space=pl.ANY),
        ],
        out_specs=pl.BlockSpec((bs, nout), lambda p, i: (i, 0)),
        out_shape=jax.ShapeDtypeStruct((n, nout), jnp.float32),
        scratch_shapes=[
            pltpu.VMEM((n, nhid), jnp.float32),
            pltpu.VMEM((n, nhid), jnp.float32),
            pltpu.VMEM((k, bs, n), jnp.float32),
            pltpu.SemaphoreType.DMA((k,)),
        ],
        compiler_params=pltpu.CompilerParams(
            dimension_semantics=("arbitrary", "arbitrary"),
        ),
    )(x, W1, b1r, W2, b2r, adj)
